# baseline (speedup 1.0000x reference)
import functools

import jax
import jax.numpy as jnp
from jax import lax
from jax.experimental import pallas as pl
from jax.experimental.pallas import tpu as pltpu
from jax.experimental.pallas import tpu_sc as plsc

_ALPHA = 0.5
_LANES = 16
_W = 128
_BLK = 16


def _degree_kernel(np_, rows):
    mesh = plsc.VectorSubcoreMesh(core_axis_name="c", subcore_axis_name="s")
    base = rows // 32
    zchunk = np_ // 16

    @functools.partial(
        pl.kernel,
        out_type=jax.ShapeDtypeStruct((2, 2, np_), jnp.float32),
        mesh=mesh,
        scratch_types=[
            pltpu.VMEM((2, base, _W), jnp.int32),
            pltpu.VMEM((_W,), jnp.float32),
            pltpu.VMEM((zchunk,), jnp.float32),
            pltpu.VMEM_SHARED((np_,), jnp.float32),
            pltpu.VMEM_SHARED((np_,), jnp.float32),
            pltpu.SemaphoreType.DMA,
        ],
    )
    def deg_k(ei_hbm, out_hbm, idx_v, ones_v, zero_v, hist_s, hist_d, sem_s):
        c = lax.axis_index("c")
        s = lax.axis_index("s")
        w = c * 16 + s

        def fill_ones(i, _):
            ones_v[pl.ds(i * _LANES, _LANES)] = jnp.ones((_LANES,), jnp.float32)
            return 0

        lax.fori_loop(0, _W // _LANES, fill_ones, 0)

        def fill_zero(i, _):
            zero_v[pl.ds(i * _LANES, _LANES)] = jnp.zeros((_LANES,), jnp.float32)
            return 0

        lax.fori_loop(0, zchunk // _LANES, fill_zero, 0)

        pltpu.sync_copy(zero_v, hist_s.at[pl.ds(s * zchunk, zchunk)])
        pltpu.sync_copy(zero_v, hist_d.at[pl.ds(s * zchunk, zchunk)])
        plsc.subcore_barrier()

        rowbase = w * base
        pltpu.sync_copy(
            ei_hbm.at[pl.ds(0, 2), pl.ds(rowbase, base)], idx_v
        )

        k = 8
        def body(r0, _):
            for j in range(k // 2):
                pltpu.async_copy(
                    ones_v, hist_s.at[idx_v.at[0, r0 * (k // 2) + j]], sem_s, add=True
                )
                pltpu.async_copy(
                    ones_v, hist_d.at[idx_v.at[1, r0 * (k // 2) + j]], sem_s, add=True
                )
            for j in range(k):
                pltpu.make_async_copy(ones_v, hist_s.at[idx_v.at[0, 0]], sem_s).wait()
            return 0

        lax.fori_loop(0, base // (k // 2), body, 0)
        plsc.subcore_barrier()

        @pl.when(s == 0)
        def _():
            pltpu.sync_copy(hist_s, out_hbm.at[c, 0])

        @pl.when(s == 1)
        def _():
            pltpu.sync_copy(hist_d, out_hbm.at[c, 1])

    return deg_k


def _prep_kernel(np_, d):
    def body(deg_ref, x_ref, xin_ref, xout_ref, opow_ref, ipow_ref):
        out_deg = deg_ref[0, 0] + deg_ref[1, 0]
        in_deg = deg_ref[0, 1] + deg_ref[1, 1]
        opow = jnp.where(out_deg > 0, lax.rsqrt(lax.sqrt(out_deg)), 0.0)
        ipow = jnp.where(in_deg > 0, lax.rsqrt(lax.sqrt(in_deg)), 0.0)
        opow_ref[...] = opow
        ipow_ref[...] = ipow
        x = x_ref[...]
        xin_ref[...] = ipow * x
        xout_ref[...] = opow * x

    nblk = 8
    gb = np_ // nblk
    return pl.pallas_call(
        body,
        grid=(nblk,),
        in_specs=[
            pl.BlockSpec((2, 2, gb, 1), lambda i: (0, 0, i, 0)),
            pl.BlockSpec((gb, d), lambda i: (i, 0)),
        ],
        out_specs=[
            pl.BlockSpec((gb, d), lambda i: (i, 0)),
            pl.BlockSpec((gb, d), lambda i: (i, 0)),
            pl.BlockSpec((gb, 1), lambda i: (i, 0)),
            pl.BlockSpec((gb, 1), lambda i: (i, 0)),
        ],
        out_shape=[
            jax.ShapeDtypeStruct((np_, d), jnp.float32),
            jax.ShapeDtypeStruct((np_, d), jnp.float32),
            jax.ShapeDtypeStruct((np_, 1), jnp.float32),
            jax.ShapeDtypeStruct((np_, 1), jnp.float32),
        ],
    )


def _spmm_kernel(np_, rows):
    mesh = plsc.VectorSubcoreMesh(core_axis_name="c", subcore_axis_name="s")
    base = rows // 16
    zchunk = np_ // 16

    @functools.partial(
        pl.kernel,
        out_type=[
            jax.ShapeDtypeStruct((np_, _W), jnp.float32),
            jax.ShapeDtypeStruct((np_, _W), jnp.float32),
        ],
        mesh=mesh,
        scratch_types=[
            pltpu.VMEM((2, 2, _BLK, _W), jnp.int32),
            pltpu.VMEM((2, _W, _W), jnp.float32),
            pltpu.VMEM_SHARED((np_, _W), jnp.float32),
            pltpu.SemaphoreType.DMA,
            pltpu.SemaphoreType.DMA,
            pltpu.SemaphoreType.DMA,
            pltpu.SemaphoreType.DMA,
            pltpu.SemaphoreType.DMA,
        ],
    )
    def spmm_k(
        xin_hbm, xout_hbm, ei_hbm, y_hbm, yt_hbm, ibuf, rbuf, acc,
        sem_a, sem_b, sem_i, ssem_a, ssem_b
    ):
        c = lax.axis_index("c")
        s = lax.axis_index("s")

        def fz(i, _):
            for j in range(_W // _LANES):
                rbuf[0, i, pl.ds(j * _LANES, _LANES)] = jnp.zeros(
                    (_LANES,), jnp.float32
                )
            return 0

        lax.fori_loop(0, _W, fz, 0)
        for j in range(zchunk // _W):
            pltpu.sync_copy(rbuf.at[0], acc.at[pl.ds(s * zchunk + j * _W, _W)])
        plsc.subcore_barrier()

        def run(table_hbm, gd, sd):
            rowbase = s * base
            nb = base // _BLK

            def idx_copy(b, slot):
                return pltpu.make_async_copy(
                    ei_hbm.at[pl.ds(0, 2), pl.ds(rowbase + b * _BLK, _BLK)],
                    ibuf.at[slot],
                    sem_i,
                )

            def g_start(bslot, wi, p, sem):
                pltpu.async_copy(
                    table_hbm.at[ibuf.at[bslot, gd, wi]], rbuf.at[p], sem
                )

            def g_wait(p, sem):
                pltpu.make_async_copy(
                    table_hbm.at[ibuf.at[0, gd, 0]], rbuf.at[p], sem
                ).wait()

            def s_start(bslot, wi, p, ssem):
                pltpu.async_copy(
                    rbuf.at[p], acc.at[ibuf.at[bslot, sd, wi]], ssem, add=True
                )

            def s_wait(p, ssem):
                pltpu.make_async_copy(
                    rbuf.at[p], acc.at[ibuf.at[0, sd, 0]], ssem
                ).wait()

            idx_copy(0, 0).start()
            idx_copy(0, 0).wait()
            g_start(0, 0, 0, sem_a)

            def block(b, _):
                cur = lax.rem(b, 2)
                nxt = 1 - cur

                for wi in range(_BLK):
                    p = wi % 2
                    sem = sem_a if p == 0 else sem_b
                    osem = ssem_b if p == 0 else ssem_a
                    ssem = ssem_a if p == 0 else ssem_b
                    g_wait(p, sem)
                    s_start(cur, wi, p, ssem)
                    if wi == 0:

                        @pl.when(b > 0)
                        def _():
                            s_wait(1, osem)

                        @pl.when(b + 1 < nb)
                        def _():
                            idx_copy(b + 1, nxt).start()

                    else:
                        s_wait(1 - p, osem)
                    if wi < _BLK - 1:
                        g_start(cur, wi + 1, 1 - p, sem_b if p == 0 else sem_a)
                    else:

                        @pl.when(b + 1 < nb)
                        def _():
                            idx_copy(0, nxt).wait()
                            g_start(nxt, 0, 1 - p, sem_b if p == 0 else sem_a)

                return 0

            lax.fori_loop(0, nb, block, 0)
            s_wait(1, ssem_b)

        @pl.when(c == 0)
        def _():
            run(xin_hbm, 1, 0)

        @pl.when(c == 1)
        def _():
            run(xout_hbm, 0, 1)

        plsc.subcore_barrier()

        def copy_out(out_hbm):
            pltpu.sync_copy(
                acc.at[pl.ds(s * zchunk, zchunk)],
                out_hbm.at[pl.ds(s * zchunk, zchunk)],
            )

        @pl.when(c == 0)
        def _():
            copy_out(y_hbm)

        @pl.when(c == 1)
        def _():
            copy_out(yt_hbm)

    return spmm_k


def _out_kernel(n, np_, d):
    def body(y_ref, yt_ref, op_ref, ip_ref, ws_ref, wd_ref, b_ref, out_ref):
        yp = op_ref[...] * y_ref[...]
        ytp = ip_ref[...] * yt_ref[...]
        dn = (((1,), (1,)), ((), ()))
        a = lax.dot_general(yp, ws_ref[...], dn, preferred_element_type=jnp.float32)
        bt = lax.dot_general(ytp, wd_ref[...], dn, preferred_element_type=jnp.float32)
        bias = _ALPHA * b_ref[0:1, :] + (1.0 - _ALPHA) * b_ref[1:2, :]
        out_ref[...] = _ALPHA * a + (1.0 - _ALPHA) * bt + bias

    nblk = 5
    gb = n // nblk
    assert n % nblk == 0 and gb % 8 == 0
    return pl.pallas_call(
        body,
        grid=(nblk,),
        in_specs=[
            pl.BlockSpec((gb, d), lambda i: (i, 0)),
            pl.BlockSpec((gb, d), lambda i: (i, 0)),
            pl.BlockSpec((gb, 1), lambda i: (i, 0)),
            pl.BlockSpec((gb, 1), lambda i: (i, 0)),
            pl.BlockSpec((d, d), lambda i: (0, 0)),
            pl.BlockSpec((d, d), lambda i: (0, 0)),
            pl.BlockSpec((2, d), lambda i: (0, 0)),
        ],
        out_specs=pl.BlockSpec((gb, d), lambda i: (i, 0)),
        out_shape=jax.ShapeDtypeStruct((n, d), jnp.float32),
    )


def kernel(x, edge_index, W_s2d, b_s2d, W_d2s, b_d2s):
    n, d = x.shape
    e = edge_index.shape[1]
    np_ = -(-(n + 1) // 2048) * 2048
    npad = np_ - n
    erows = -(-e // (256 * _W)) * 256
    epad = erows * _W - e

    padvals = n + (jnp.arange(epad, dtype=jnp.int32) % npad)
    ei3 = jnp.concatenate(
        [edge_index, jnp.stack([padvals, padvals])], axis=1
    ).reshape(2, erows, _W)
    x_p = jnp.concatenate([x, jnp.zeros((npad, d), jnp.float32)], axis=0)

    deg = _degree_kernel(np_, erows)(ei3)
    x_in, x_out, opow, ipow = _prep_kernel(np_, d)(deg.reshape(2, 2, np_, 1), x_p)
    yy, yt = _spmm_kernel(np_, erows)(x_in, x_out, ei3)
    b2 = jnp.stack([b_s2d, b_d2s])
    return _out_kernel(n, np_, d)(yy, yt, opow, ipow, W_s2d, W_d2s, b2)


# speedup vs baseline: 1.1633x; 1.1633x over previous
import functools

import jax
import jax.numpy as jnp
from jax import lax
from jax.experimental import pallas as pl
from jax.experimental.pallas import tpu as pltpu
from jax.experimental.pallas import tpu_sc as plsc

_ALPHA = 0.5
_LANES = 16
_W = 128
_BLK = 16


def _degree_kernel(np_, rows):
    mesh = plsc.VectorSubcoreMesh(core_axis_name="c", subcore_axis_name="s")
    base = rows // 32
    zchunk = np_ // 16

    @functools.partial(
        pl.kernel,
        out_type=jax.ShapeDtypeStruct((2, 2, np_), jnp.float32),
        mesh=mesh,
        scratch_types=[
            pltpu.VMEM((2, base, _W), jnp.int32),
            pltpu.VMEM((_W,), jnp.float32),
            pltpu.VMEM((zchunk,), jnp.float32),
            pltpu.VMEM_SHARED((np_,), jnp.float32),
            pltpu.VMEM_SHARED((np_,), jnp.float32),
            pltpu.SemaphoreType.DMA,
        ],
    )
    def deg_k(ei_hbm, out_hbm, idx_v, ones_v, zero_v, hist_s, hist_d, sem_s):
        c = lax.axis_index("c")
        s = lax.axis_index("s")
        w = c * 16 + s

        def fill_ones(i, _):
            ones_v[pl.ds(i * _LANES, _LANES)] = jnp.ones((_LANES,), jnp.float32)
            return 0

        lax.fori_loop(0, _W // _LANES, fill_ones, 0)

        def fill_zero(i, _):
            zero_v[pl.ds(i * _LANES, _LANES)] = jnp.zeros((_LANES,), jnp.float32)
            return 0

        lax.fori_loop(0, zchunk // _LANES, fill_zero, 0)

        pltpu.sync_copy(zero_v, hist_s.at[pl.ds(s * zchunk, zchunk)])
        pltpu.sync_copy(zero_v, hist_d.at[pl.ds(s * zchunk, zchunk)])
        plsc.subcore_barrier()

        rowbase = w * base
        pltpu.sync_copy(
            ei_hbm.at[pl.ds(0, 2), pl.ds(rowbase, base)], idx_v
        )

        k = 8
        def body(r0, _):
            for j in range(k // 2):
                pltpu.async_copy(
                    ones_v, hist_s.at[idx_v.at[0, r0 * (k // 2) + j]], sem_s, add=True
                )
                pltpu.async_copy(
                    ones_v, hist_d.at[idx_v.at[1, r0 * (k // 2) + j]], sem_s, add=True
                )
            for j in range(k):
                pltpu.make_async_copy(ones_v, hist_s.at[idx_v.at[0, 0]], sem_s).wait()
            return 0

        lax.fori_loop(0, base // (k // 2), body, 0)
        plsc.subcore_barrier()

        @pl.when(s == 0)
        def _():
            pltpu.sync_copy(hist_s, out_hbm.at[c, 0])

        @pl.when(s == 1)
        def _():
            pltpu.sync_copy(hist_d, out_hbm.at[c, 1])

    return deg_k


def _prep_kernel(np_, d):
    def body(deg_ref, x_ref, xin_ref, xout_ref, opow_ref, ipow_ref):
        out_deg = deg_ref[0, 0] + deg_ref[1, 0]
        in_deg = deg_ref[0, 1] + deg_ref[1, 1]
        opow = jnp.where(out_deg > 0, lax.rsqrt(lax.sqrt(out_deg)), 0.0)
        ipow = jnp.where(in_deg > 0, lax.rsqrt(lax.sqrt(in_deg)), 0.0)
        opow_ref[...] = opow
        ipow_ref[...] = ipow
        x = x_ref[...]
        xin_ref[...] = ipow * x
        xout_ref[...] = opow * x

    nblk = 8
    gb = np_ // nblk
    return pl.pallas_call(
        body,
        grid=(nblk,),
        in_specs=[
            pl.BlockSpec((2, 2, gb, 1), lambda i: (0, 0, i, 0)),
            pl.BlockSpec((gb, d), lambda i: (i, 0)),
        ],
        out_specs=[
            pl.BlockSpec((gb, d), lambda i: (i, 0)),
            pl.BlockSpec((gb, d), lambda i: (i, 0)),
            pl.BlockSpec((gb, 1), lambda i: (i, 0)),
            pl.BlockSpec((gb, 1), lambda i: (i, 0)),
        ],
        out_shape=[
            jax.ShapeDtypeStruct((np_, d), jnp.float32),
            jax.ShapeDtypeStruct((np_, d), jnp.float32),
            jax.ShapeDtypeStruct((np_, 1), jnp.float32),
            jax.ShapeDtypeStruct((np_, 1), jnp.float32),
        ],
    )


def _spmm_kernel(np_, rows):
    mesh = plsc.VectorSubcoreMesh(core_axis_name="c", subcore_axis_name="s")
    base = rows // 16
    zchunk = np_ // 16

    @functools.partial(
        pl.kernel,
        out_type=[
            jax.ShapeDtypeStruct((np_, _W), jnp.float32),
            jax.ShapeDtypeStruct((np_, _W), jnp.float32),
        ],
        mesh=mesh,
        scratch_types=[
            pltpu.VMEM((2, 2, _BLK, _W), jnp.int32),
            pltpu.VMEM((2, _W, _W), jnp.float32),
            pltpu.VMEM_SHARED((np_, _W), jnp.float32),
            pltpu.SemaphoreType.DMA,
            pltpu.SemaphoreType.DMA,
            pltpu.SemaphoreType.DMA,
        ],
    )
    def spmm_k(xin_hbm, xout_hbm, ei_hbm, y_hbm, yt_hbm, ibuf, rbuf, acc, sem_a, sem_b, sem_i):
        c = lax.axis_index("c")
        s = lax.axis_index("s")

        def fz(i, _):
            for j in range(_W // _LANES):
                rbuf[0, i, pl.ds(j * _LANES, _LANES)] = jnp.zeros(
                    (_LANES,), jnp.float32
                )
            return 0

        lax.fori_loop(0, _W, fz, 0)
        for j in range(zchunk // _W):
            pltpu.sync_copy(rbuf.at[0], acc.at[pl.ds(s * zchunk + j * _W, _W)])
        plsc.subcore_barrier()

        def run(table_hbm, gd, sd):
            rowbase = s * base
            nb = base // _BLK

            def idx_copy(b, slot):
                return pltpu.make_async_copy(
                    ei_hbm.at[pl.ds(0, 2), pl.ds(rowbase + b * _BLK, _BLK)],
                    ibuf.at[slot],
                    sem_i,
                )

            def g_start(bslot, wi, p, sem):
                pltpu.async_copy(
                    table_hbm.at[ibuf.at[bslot, gd, wi]], rbuf.at[p], sem
                )

            def g_wait(p, sem):
                pltpu.make_async_copy(
                    table_hbm.at[ibuf.at[0, gd, 0]], rbuf.at[p], sem
                ).wait()

            idx_copy(0, 0).start()
            idx_copy(0, 0).wait()
            g_start(0, 0, 0, sem_a)
            g_start(0, 1, 1, sem_b)

            def block(b, _):
                cur = lax.rem(b, 2)
                nxt = 1 - cur

                @pl.when(b + 1 < nb)
                def _():
                    idx_copy(b + 1, nxt).start()

                for wi in range(_BLK):
                    p = wi % 2
                    sem = sem_a if p == 0 else sem_b
                    g_wait(p, sem)
                    pltpu.sync_copy(rbuf.at[p], acc.at[ibuf.at[cur, sd, wi]], add=True)
                    if wi < _BLK - 2:
                        g_start(cur, wi + 2, p, sem)
                    else:

                        @pl.when(b + 1 < nb)
                        def _(wi=wi, p=p, sem=sem):
                            if wi == _BLK - 2:
                                idx_copy(0, nxt).wait()
                            g_start(nxt, wi - (_BLK - 2), p, sem)

                return 0

            lax.fori_loop(0, nb, block, 0)

        @pl.when(c == 0)
        def _():
            run(xin_hbm, 1, 0)

        @pl.when(c == 1)
        def _():
            run(xout_hbm, 0, 1)

        plsc.subcore_barrier()

        def copy_out(out_hbm):
            pltpu.sync_copy(
                acc.at[pl.ds(s * zchunk, zchunk)],
                out_hbm.at[pl.ds(s * zchunk, zchunk)],
            )

        @pl.when(c == 0)
        def _():
            copy_out(y_hbm)

        @pl.when(c == 1)
        def _():
            copy_out(yt_hbm)

    return spmm_k


def _out_kernel(n, np_, d):
    def body(y_ref, yt_ref, op_ref, ip_ref, ws_ref, wd_ref, b_ref, out_ref):
        yp = op_ref[...] * y_ref[...]
        ytp = ip_ref[...] * yt_ref[...]
        dn = (((1,), (1,)), ((), ()))
        a = lax.dot_general(yp, ws_ref[...], dn, preferred_element_type=jnp.float32)
        bt = lax.dot_general(ytp, wd_ref[...], dn, preferred_element_type=jnp.float32)
        bias = _ALPHA * b_ref[0:1, :] + (1.0 - _ALPHA) * b_ref[1:2, :]
        out_ref[...] = _ALPHA * a + (1.0 - _ALPHA) * bt + bias

    nblk = 5
    gb = n // nblk
    assert n % nblk == 0 and gb % 8 == 0
    return pl.pallas_call(
        body,
        grid=(nblk,),
        in_specs=[
            pl.BlockSpec((gb, d), lambda i: (i, 0)),
            pl.BlockSpec((gb, d), lambda i: (i, 0)),
            pl.BlockSpec((gb, 1), lambda i: (i, 0)),
            pl.BlockSpec((gb, 1), lambda i: (i, 0)),
            pl.BlockSpec((d, d), lambda i: (0, 0)),
            pl.BlockSpec((d, d), lambda i: (0, 0)),
            pl.BlockSpec((2, d), lambda i: (0, 0)),
        ],
        out_specs=pl.BlockSpec((gb, d), lambda i: (i, 0)),
        out_shape=jax.ShapeDtypeStruct((n, d), jnp.float32),
    )


def kernel(x, edge_index, W_s2d, b_s2d, W_d2s, b_d2s):
    n, d = x.shape
    e = edge_index.shape[1]
    np_ = -(-(n + 1) // 2048) * 2048
    npad = np_ - n
    erows = -(-e // (256 * _W)) * 256
    epad = erows * _W - e

    padvals = n + (jnp.arange(epad, dtype=jnp.int32) % npad)
    ei3 = jnp.concatenate(
        [edge_index, jnp.stack([padvals, padvals])], axis=1
    ).reshape(2, erows, _W)
    x_p = jnp.concatenate([x, jnp.zeros((npad, d), jnp.float32)], axis=0)

    deg = _degree_kernel(np_, erows)(ei3)
    x_in, x_out, opow, ipow = _prep_kernel(np_, d)(deg.reshape(2, 2, np_, 1), x_p)
    yy, yt = _spmm_kernel(np_, erows)(x_in, x_out, ei3)
    b2 = jnp.stack([b_s2d, b_d2s])
    return _out_kernel(n, np_, d)(yy, yt, opow, ipow, W_s2d, W_d2s, b2)


# speedup vs baseline: 1.1724x; 1.0078x over previous
import functools

import jax
import jax.numpy as jnp
from jax import lax
from jax.experimental import pallas as pl
from jax.experimental.pallas import tpu as pltpu
from jax.experimental.pallas import tpu_sc as plsc

_ALPHA = 0.5
_LANES = 16
_W = 128
_BLK = 16


def _degree_kernel(np_, rows):
    mesh = plsc.VectorSubcoreMesh(core_axis_name="c", subcore_axis_name="s")
    base = rows // 32
    zchunk = np_ // 16

    @functools.partial(
        pl.kernel,
        out_type=jax.ShapeDtypeStruct((2, 2, np_), jnp.float32),
        mesh=mesh,
        scratch_types=[
            pltpu.VMEM((2, base, _W), jnp.int32),
            pltpu.VMEM((_W,), jnp.float32),
            pltpu.VMEM((zchunk,), jnp.float32),
            pltpu.VMEM_SHARED((np_,), jnp.float32),
            pltpu.VMEM_SHARED((np_,), jnp.float32),
            pltpu.SemaphoreType.DMA,
        ],
    )
    def deg_k(ei_hbm, out_hbm, idx_v, ones_v, zero_v, hist_s, hist_d, sem_s):
        c = lax.axis_index("c")
        s = lax.axis_index("s")
        w = c * 16 + s

        def fill_ones(i, _):
            ones_v[pl.ds(i * _LANES, _LANES)] = jnp.ones((_LANES,), jnp.float32)
            return 0

        lax.fori_loop(0, _W // _LANES, fill_ones, 0)

        def fill_zero(i, _):
            zero_v[pl.ds(i * _LANES, _LANES)] = jnp.zeros((_LANES,), jnp.float32)
            return 0

        lax.fori_loop(0, zchunk // _LANES, fill_zero, 0)

        pltpu.sync_copy(zero_v, hist_s.at[pl.ds(s * zchunk, zchunk)])
        pltpu.sync_copy(zero_v, hist_d.at[pl.ds(s * zchunk, zchunk)])
        plsc.subcore_barrier()

        rowbase = w * base
        pltpu.sync_copy(
            ei_hbm.at[pl.ds(0, 2), pl.ds(rowbase, base)], idx_v
        )

        k = 16
        def body(r0, _):
            for j in range(k // 2):
                pltpu.async_copy(
                    ones_v, hist_s.at[idx_v.at[0, r0 * (k // 2) + j]], sem_s, add=True
                )
                pltpu.async_copy(
                    ones_v, hist_d.at[idx_v.at[1, r0 * (k // 2) + j]], sem_s, add=True
                )
            for j in range(k):
                pltpu.make_async_copy(ones_v, hist_s.at[idx_v.at[0, 0]], sem_s).wait()
            return 0

        lax.fori_loop(0, base // (k // 2), body, 0)
        plsc.subcore_barrier()

        @pl.when(s == 0)
        def _():
            pltpu.sync_copy(hist_s, out_hbm.at[c, 0])

        @pl.when(s == 1)
        def _():
            pltpu.sync_copy(hist_d, out_hbm.at[c, 1])

    return deg_k


def _prep_kernel(n, np_, d):
    def body(deg_ref, x_ref, xin_ref, xout_ref, opow_ref, ipow_ref):
        out_deg = deg_ref[0, 0] + deg_ref[1, 0]
        in_deg = deg_ref[0, 1] + deg_ref[1, 1]
        opow = jnp.where(out_deg > 0, lax.rsqrt(lax.sqrt(out_deg)), 0.0)
        ipow = jnp.where(in_deg > 0, lax.rsqrt(lax.sqrt(in_deg)), 0.0)
        opow_ref[...] = opow
        ipow_ref[...] = ipow
        x = x_ref[...]
        xin_ref[...] = ipow * x
        xout_ref[...] = opow * x

    nblk = 5
    gb = n // nblk
    assert n % nblk == 0 and gb % 8 == 0
    return pl.pallas_call(
        body,
        grid=(nblk,),
        in_specs=[
            pl.BlockSpec((2, 2, gb, 1), lambda i: (0, 0, i, 0)),
            pl.BlockSpec((gb, d), lambda i: (i, 0)),
        ],
        out_specs=[
            pl.BlockSpec((gb, d), lambda i: (i, 0)),
            pl.BlockSpec((gb, d), lambda i: (i, 0)),
            pl.BlockSpec((gb, 1), lambda i: (i, 0)),
            pl.BlockSpec((gb, 1), lambda i: (i, 0)),
        ],
        out_shape=[
            jax.ShapeDtypeStruct((np_, d), jnp.float32),
            jax.ShapeDtypeStruct((np_, d), jnp.float32),
            jax.ShapeDtypeStruct((np_, 1), jnp.float32),
            jax.ShapeDtypeStruct((np_, 1), jnp.float32),
        ],
    )


def _spmm_kernel(np_, rows):
    mesh = plsc.VectorSubcoreMesh(core_axis_name="c", subcore_axis_name="s")
    base = rows // 16
    zchunk = np_ // 16

    @functools.partial(
        pl.kernel,
        out_type=[
            jax.ShapeDtypeStruct((np_, _W), jnp.float32),
            jax.ShapeDtypeStruct((np_, _W), jnp.float32),
        ],
        mesh=mesh,
        scratch_types=[
            pltpu.VMEM((2, 2, _BLK, _W), jnp.int32),
            pltpu.VMEM((2, _W, _W), jnp.float32),
            pltpu.VMEM_SHARED((np_, _W), jnp.float32),
            pltpu.SemaphoreType.DMA,
            pltpu.SemaphoreType.DMA,
            pltpu.SemaphoreType.DMA,
        ],
    )
    def spmm_k(xin_hbm, xout_hbm, ei_hbm, y_hbm, yt_hbm, ibuf, rbuf, acc, sem_a, sem_b, sem_i):
        c = lax.axis_index("c")
        s = lax.axis_index("s")

        def fz(i, _):
            for j in range(_W // _LANES):
                rbuf[0, i, pl.ds(j * _LANES, _LANES)] = jnp.zeros(
                    (_LANES,), jnp.float32
                )
            return 0

        lax.fori_loop(0, _W, fz, 0)
        for j in range(zchunk // _W):
            pltpu.sync_copy(rbuf.at[0], acc.at[pl.ds(s * zchunk + j * _W, _W)])
        plsc.subcore_barrier()

        def run(table_hbm, gd, sd):
            rowbase = s * base
            nb = base // _BLK

            def idx_copy(b, slot):
                return pltpu.make_async_copy(
                    ei_hbm.at[pl.ds(0, 2), pl.ds(rowbase + b * _BLK, _BLK)],
                    ibuf.at[slot],
                    sem_i,
                )

            def g_start(bslot, wi, p, sem):
                pltpu.async_copy(
                    table_hbm.at[ibuf.at[bslot, gd, wi]], rbuf.at[p], sem
                )

            def g_wait(p, sem):
                pltpu.make_async_copy(
                    table_hbm.at[ibuf.at[0, gd, 0]], rbuf.at[p], sem
                ).wait()

            idx_copy(0, 0).start()
            idx_copy(0, 0).wait()
            g_start(0, 0, 0, sem_a)
            g_start(0, 1, 1, sem_b)

            def block(b, _):
                cur = lax.rem(b, 2)
                nxt = 1 - cur

                @pl.when(b + 1 < nb)
                def _():
                    idx_copy(b + 1, nxt).start()

                for wi in range(_BLK):
                    p = wi % 2
                    sem = sem_a if p == 0 else sem_b
                    g_wait(p, sem)
                    pltpu.sync_copy(rbuf.at[p], acc.at[ibuf.at[cur, sd, wi]], add=True)
                    if wi < _BLK - 2:
                        g_start(cur, wi + 2, p, sem)
                    else:

                        @pl.when(b + 1 < nb)
                        def _(wi=wi, p=p, sem=sem):
                            if wi == _BLK - 2:
                                idx_copy(0, nxt).wait()
                            g_start(nxt, wi - (_BLK - 2), p, sem)

                return 0

            lax.fori_loop(0, nb, block, 0)

        @pl.when(c == 0)
        def _():
            run(xin_hbm, 1, 0)

        @pl.when(c == 1)
        def _():
            run(xout_hbm, 0, 1)

        plsc.subcore_barrier()

        def copy_out(out_hbm):
            pltpu.sync_copy(
                acc.at[pl.ds(s * zchunk, zchunk)],
                out_hbm.at[pl.ds(s * zchunk, zchunk)],
            )

        @pl.when(c == 0)
        def _():
            copy_out(y_hbm)

        @pl.when(c == 1)
        def _():
            copy_out(yt_hbm)

    return spmm_k


def _out_kernel(n, np_, d):
    def body(y_ref, yt_ref, op_ref, ip_ref, ws_ref, wd_ref, b_ref, out_ref):
        yp = op_ref[...] * y_ref[...]
        ytp = ip_ref[...] * yt_ref[...]
        dn = (((1,), (1,)), ((), ()))
        a = lax.dot_general(yp, ws_ref[...], dn, preferred_element_type=jnp.float32)
        bt = lax.dot_general(ytp, wd_ref[...], dn, preferred_element_type=jnp.float32)
        bias = _ALPHA * b_ref[0:1, :] + (1.0 - _ALPHA) * b_ref[1:2, :]
        out_ref[...] = _ALPHA * a + (1.0 - _ALPHA) * bt + bias

    nblk = 5
    gb = n // nblk
    assert n % nblk == 0 and gb % 8 == 0
    return pl.pallas_call(
        body,
        grid=(nblk,),
        in_specs=[
            pl.BlockSpec((gb, d), lambda i: (i, 0)),
            pl.BlockSpec((gb, d), lambda i: (i, 0)),
            pl.BlockSpec((gb, 1), lambda i: (i, 0)),
            pl.BlockSpec((gb, 1), lambda i: (i, 0)),
            pl.BlockSpec((d, d), lambda i: (0, 0)),
            pl.BlockSpec((d, d), lambda i: (0, 0)),
            pl.BlockSpec((2, d), lambda i: (0, 0)),
        ],
        out_specs=pl.BlockSpec((gb, d), lambda i: (i, 0)),
        out_shape=jax.ShapeDtypeStruct((n, d), jnp.float32),
    )


def kernel(x, edge_index, W_s2d, b_s2d, W_d2s, b_d2s):
    n, d = x.shape
    e = edge_index.shape[1]
    np_ = -(-(n + 1) // 2048) * 2048
    npad = np_ - n
    erows = -(-e // (256 * _W)) * 256
    epad = erows * _W - e

    padvals = n + (jnp.arange(epad, dtype=jnp.int32) % npad)
    ei3 = jnp.concatenate(
        [edge_index, jnp.stack([padvals, padvals])], axis=1
    ).reshape(2, erows, _W)
    deg = _degree_kernel(np_, erows)(ei3)
    x_in, x_out, opow, ipow = _prep_kernel(n, np_, d)(deg.reshape(2, 2, np_, 1), x)
    yy, yt = _spmm_kernel(np_, erows)(x_in, x_out, ei3)
    b2 = jnp.stack([b_s2d, b_d2s])
    return _out_kernel(n, np_, d)(yy, yt, opow, ipow, W_s2d, W_d2s, b2)


# speedup vs baseline: 1.1775x; 1.0044x over previous
import functools

import jax
import jax.numpy as jnp
from jax import lax
from jax.experimental import pallas as pl
from jax.experimental.pallas import tpu as pltpu
from jax.experimental.pallas import tpu_sc as plsc

_ALPHA = 0.5
_LANES = 16
_W = 128
_BLK = 16


def _degree_kernel(np_, rows):
    mesh = plsc.VectorSubcoreMesh(core_axis_name="c", subcore_axis_name="s")
    base = rows // 32
    zchunk = np_ // 16

    @functools.partial(
        pl.kernel,
        out_type=jax.ShapeDtypeStruct((2, 2, np_), jnp.float32),
        mesh=mesh,
        scratch_types=[
            pltpu.VMEM((2, base, _W), jnp.int32),
            pltpu.VMEM((_W,), jnp.float32),
            pltpu.VMEM((zchunk,), jnp.float32),
            pltpu.VMEM_SHARED((np_,), jnp.float32),
            pltpu.VMEM_SHARED((np_,), jnp.float32),
            pltpu.SemaphoreType.DMA,
        ],
    )
    def deg_k(ei_hbm, out_hbm, idx_v, ones_v, zero_v, hist_s, hist_d, sem_s):
        c = lax.axis_index("c")
        s = lax.axis_index("s")
        w = c * 16 + s

        def fill_ones(i, _):
            ones_v[pl.ds(i * _LANES, _LANES)] = jnp.ones((_LANES,), jnp.float32)
            return 0

        lax.fori_loop(0, _W // _LANES, fill_ones, 0)

        def fill_zero(i, _):
            zero_v[pl.ds(i * _LANES, _LANES)] = jnp.zeros((_LANES,), jnp.float32)
            return 0

        lax.fori_loop(0, zchunk // _LANES, fill_zero, 0)

        pltpu.sync_copy(zero_v, hist_s.at[pl.ds(s * zchunk, zchunk)])
        pltpu.sync_copy(zero_v, hist_d.at[pl.ds(s * zchunk, zchunk)])
        plsc.subcore_barrier()

        rowbase = w * base
        pltpu.sync_copy(
            ei_hbm.at[pl.ds(0, 2), pl.ds(rowbase, base)], idx_v
        )

        k = 16
        def body(r0, _):
            for j in range(k // 2):
                pltpu.async_copy(
                    ones_v, hist_s.at[idx_v.at[0, r0 * (k // 2) + j]], sem_s, add=True
                )
                pltpu.async_copy(
                    ones_v, hist_d.at[idx_v.at[1, r0 * (k // 2) + j]], sem_s, add=True
                )
            for j in range(k):
                pltpu.make_async_copy(ones_v, hist_s.at[idx_v.at[0, 0]], sem_s).wait()
            return 0

        lax.fori_loop(0, base // (k // 2), body, 0)
        plsc.subcore_barrier()

        @pl.when(s == 0)
        def _():
            pltpu.sync_copy(hist_s, out_hbm.at[c, 0])

        @pl.when(s == 1)
        def _():
            pltpu.sync_copy(hist_d, out_hbm.at[c, 1])

    return deg_k


def _prep_kernel(n, np_, d):
    def body(deg_ref, x_ref, xin_ref, xout_ref, opow_ref, ipow_ref):
        out_deg = deg_ref[0, 0] + deg_ref[1, 0]
        in_deg = deg_ref[0, 1] + deg_ref[1, 1]
        opow = jnp.where(out_deg > 0, lax.rsqrt(lax.sqrt(out_deg)), 0.0)
        ipow = jnp.where(in_deg > 0, lax.rsqrt(lax.sqrt(in_deg)), 0.0)
        opow_ref[...] = opow
        ipow_ref[...] = ipow
        x = x_ref[...]
        xin_ref[...] = ipow * x
        xout_ref[...] = opow * x

    nblk = 5
    gb = n // nblk
    assert n % nblk == 0 and gb % 8 == 0
    return pl.pallas_call(
        body,
        grid=(nblk,),
        in_specs=[
            pl.BlockSpec((2, 2, gb, 1), lambda i: (0, 0, i, 0)),
            pl.BlockSpec((gb, d), lambda i: (i, 0)),
        ],
        out_specs=[
            pl.BlockSpec((gb, d), lambda i: (i, 0)),
            pl.BlockSpec((gb, d), lambda i: (i, 0)),
            pl.BlockSpec((gb, 1), lambda i: (i, 0)),
            pl.BlockSpec((gb, 1), lambda i: (i, 0)),
        ],
        out_shape=[
            jax.ShapeDtypeStruct((np_, d), jnp.float32),
            jax.ShapeDtypeStruct((np_, d), jnp.float32),
            jax.ShapeDtypeStruct((np_, 1), jnp.float32),
            jax.ShapeDtypeStruct((np_, 1), jnp.float32),
        ],
    )


def _spmm_kernel(np_, rows):
    mesh = plsc.VectorSubcoreMesh(core_axis_name="c", subcore_axis_name="s")
    base = rows // 16
    zchunk = np_ // 16

    @functools.partial(
        pl.kernel,
        out_type=[
            jax.ShapeDtypeStruct((np_, _W), jnp.float32),
            jax.ShapeDtypeStruct((np_, _W), jnp.float32),
        ],
        mesh=mesh,
        scratch_types=[
            pltpu.VMEM((2, 2, _BLK, _W), jnp.int32),
            pltpu.VMEM((2, _W, _W), jnp.float32),
            pltpu.VMEM_SHARED((np_, _W), jnp.float32),
            pltpu.SemaphoreType.DMA,
            pltpu.SemaphoreType.DMA,
            pltpu.SemaphoreType.DMA,
        ],
    )
    def spmm_k(xin_hbm, xout_hbm, ei_hbm, y_hbm, yt_hbm, ibuf, rbuf, acc, sem_a, sem_b, sem_i):
        c = lax.axis_index("c")
        s = lax.axis_index("s")

        def fz(i, _):
            for j in range(_W // _LANES):
                rbuf[0, i, pl.ds(j * _LANES, _LANES)] = jnp.zeros(
                    (_LANES,), jnp.float32
                )
            return 0

        lax.fori_loop(0, _W, fz, 0)
        for j in range(zchunk // _W):
            pltpu.sync_copy(rbuf.at[0], acc.at[pl.ds(s * zchunk + j * _W, _W)])
        plsc.subcore_barrier()

        def run(table_hbm, gd, sd):
            rowbase = s * base
            nb = base // _BLK

            def idx_copy(b, slot):
                return pltpu.make_async_copy(
                    ei_hbm.at[pl.ds(0, 2), pl.ds(rowbase + b * _BLK, _BLK)],
                    ibuf.at[slot],
                    sem_i,
                )

            def g_start(bslot, wi, p, sem):
                pltpu.async_copy(
                    table_hbm.at[ibuf.at[bslot, gd, wi]], rbuf.at[p], sem
                )

            def g_wait(p, sem):
                pltpu.make_async_copy(
                    table_hbm.at[ibuf.at[0, gd, 0]], rbuf.at[p], sem
                ).wait()

            idx_copy(0, 0).start()
            idx_copy(0, 0).wait()
            g_start(0, 0, 0, sem_a)
            g_start(0, 1, 1, sem_b)

            def block(b, _):
                cur = lax.rem(b, 2)
                nxt = 1 - cur

                @pl.when(b + 1 < nb)
                def _():
                    idx_copy(b + 1, nxt).start()

                for wi in range(_BLK):
                    p = wi % 2
                    sem = sem_a if p == 0 else sem_b
                    g_wait(p, sem)
                    pltpu.sync_copy(rbuf.at[p], acc.at[ibuf.at[cur, sd, wi]], add=True)
                    if wi < _BLK - 2:
                        g_start(cur, wi + 2, p, sem)
                    else:

                        @pl.when(b + 1 < nb)
                        def _(wi=wi, p=p, sem=sem):
                            if wi == _BLK - 2:
                                idx_copy(0, nxt).wait()
                            g_start(nxt, wi - (_BLK - 2), p, sem)

                return 0

            lax.fori_loop(0, nb, block, 0)

        @pl.when(c == 0)
        def _():
            run(xin_hbm, 1, 0)

        @pl.when(c == 1)
        def _():
            run(xout_hbm, 0, 1)

        plsc.subcore_barrier()

        def copy_out(out_hbm):
            pltpu.sync_copy(
                acc.at[pl.ds(s * zchunk, zchunk)],
                out_hbm.at[pl.ds(s * zchunk, zchunk)],
            )

        @pl.when(c == 0)
        def _():
            copy_out(y_hbm)

        @pl.when(c == 1)
        def _():
            copy_out(yt_hbm)

    return spmm_k


def _out_kernel(n, np_, d):
    def body(y_ref, yt_ref, op_ref, ip_ref, ws_ref, wd_ref, b_ref, out_ref):
        yp = op_ref[...] * y_ref[...]
        ytp = ip_ref[...] * yt_ref[...]
        dn = (((1,), (1,)), ((), ()))
        a = lax.dot_general(yp, ws_ref[...], dn, preferred_element_type=jnp.float32)
        bt = lax.dot_general(ytp, wd_ref[...], dn, preferred_element_type=jnp.float32)
        bias = _ALPHA * b_ref[0:1, :] + (1.0 - _ALPHA) * b_ref[1:2, :]
        out_ref[...] = _ALPHA * a + (1.0 - _ALPHA) * bt + bias

    nblk = 5
    gb = n // nblk
    assert n % nblk == 0 and gb % 8 == 0
    return pl.pallas_call(
        body,
        grid=(nblk,),
        in_specs=[
            pl.BlockSpec((gb, d), lambda i: (i, 0)),
            pl.BlockSpec((gb, d), lambda i: (i, 0)),
            pl.BlockSpec((gb, 1), lambda i: (i, 0)),
            pl.BlockSpec((gb, 1), lambda i: (i, 0)),
            pl.BlockSpec((d, d), lambda i: (0, 0)),
            pl.BlockSpec((d, d), lambda i: (0, 0)),
            pl.BlockSpec((2, d), lambda i: (0, 0)),
        ],
        out_specs=pl.BlockSpec((gb, d), lambda i: (i, 0)),
        out_shape=jax.ShapeDtypeStruct((n, d), jnp.float32),
    )


def kernel(x, edge_index, W_s2d, b_s2d, W_d2s, b_d2s):
    n, d = x.shape
    e = edge_index.shape[1]
    assert d == _W and e % _W == 0
    np_ = -(-(n + 1) // 2048) * 2048
    npad = np_ - n
    erows = -(-e // (256 * _W)) * 256
    epad = erows * _W - e

    padvals = n + (jnp.arange(epad, dtype=jnp.int32) % npad)
    ei3 = jnp.concatenate(
        [edge_index, jnp.stack([padvals, padvals])], axis=1
    ).reshape(2, erows, _W)
    deg = _degree_kernel(np_, erows)(ei3)
    x_in, x_out, opow, ipow = _prep_kernel(n, np_, d)(deg.reshape(2, 2, np_, 1), x)
    yy, yt = _spmm_kernel(np_, erows)(x_in, x_out, ei3)
    b2 = jnp.stack([b_s2d, b_d2s])
    return _out_kernel(n, np_, d)(yy, yt, opow, ipow, W_s2d, W_d2s, b2)
